# baseline (speedup 1.0000x reference)
import functools

import jax
import jax.numpy as jnp
from jax import lax
from jax.experimental import pallas as pl
from jax.experimental.pallas import tpu as pltpu
from jax.experimental.pallas import tpu_sc as plsc

M = 128


def _routing(group_indices, B, G, NB):
    gi = group_indices.astype(jnp.int32)
    sort_idx = jnp.argsort(gi, stable=True).astype(jnp.int32)
    counts = jnp.zeros((G,), jnp.int32).at[gi].add(1)
    csum = jnp.cumsum(counts)
    offsets = csum - counts
    nblk = -(-counts // M)
    blk_cum = jnp.cumsum(nblk)
    blk_start = blk_cum - nblk
    i = jnp.arange(NB, dtype=jnp.int32)
    bg = jnp.searchsorted(blk_cum, i, side="right").astype(jnp.int32)
    last_g = gi[sort_idx[B - 1]]
    bg = jnp.where(bg >= G, last_g, bg)
    m = jnp.arange(M, dtype=jnp.int32)
    local = (i[:, None] - blk_start[bg][:, None]) * M + m[None, :]
    local = jnp.clip(local, 0, counts[bg][:, None] - 1)
    row_idx = sort_idx[offsets[bg][:, None] + local].reshape(NB * M)
    return row_idx, bg


def _sc_gather(x, row_idx, P, D):
    info = plsc.get_sparse_core_info()
    NC, NS = info.num_cores, info.num_subcores
    NW = NC * NS
    bpw = P // NW
    mesh = plsc.VectorSubcoreMesh(core_axis_name="c", subcore_axis_name="s")

    @functools.partial(
        pl.kernel,
        mesh=mesh,
        out_type=jax.ShapeDtypeStruct((P, D), jnp.float32),
        scratch_types=[
            pltpu.VMEM((bpw,), jnp.int32),
            pltpu.VMEM((bpw, D), jnp.float32),
            pltpu.SemaphoreType.DMA,
        ],
    )
    def k(x_hbm, idx_hbm, out_hbm, idx_v, rows_v, sem):
        wid = lax.axis_index("s") * NC + lax.axis_index("c")
        base = wid * bpw
        pltpu.sync_copy(idx_hbm.at[pl.ds(base, bpw)], idx_v)
        pltpu.async_copy(x_hbm.at[idx_v], rows_v, sem).wait()
        pltpu.sync_copy(rows_v, out_hbm.at[pl.ds(base, bpw)])

    return k(x, row_idx)


def _sc_scatter(y_sorted, row_idx, B, P, D):
    info = plsc.get_sparse_core_info()
    NC, NS = info.num_cores, info.num_subcores
    NW = NC * NS
    bpw = P // NW
    mesh = plsc.VectorSubcoreMesh(core_axis_name="c", subcore_axis_name="s")

    @functools.partial(
        pl.kernel,
        mesh=mesh,
        out_type=jax.ShapeDtypeStruct((B, D), jnp.float32),
        scratch_types=[
            pltpu.VMEM((bpw,), jnp.int32),
            pltpu.VMEM((bpw, D), jnp.float32),
            pltpu.SemaphoreType.DMA,
        ],
    )
    def k(ys_hbm, idx_hbm, out_hbm, idx_v, rows_v, sem):
        wid = lax.axis_index("s") * NC + lax.axis_index("c")
        base = wid * bpw
        pltpu.sync_copy(idx_hbm.at[pl.ds(base, bpw)], idx_v)
        pltpu.sync_copy(ys_hbm.at[pl.ds(base, bpw)], rows_v)
        pltpu.async_copy(rows_v, out_hbm.at[idx_v], sem).wait()

    return k(y_sorted, row_idx)


def _tc_grouped_matmul(x_sorted, W, b, block_group, NB, D):
    def body(bg_ref, x_ref, w_ref, b_ref, o_ref):
        o_ref[...] = (
            jnp.dot(x_ref[...], w_ref[0], preferred_element_type=jnp.float32)
            + b_ref[0]
        )

    G = W.shape[0]
    grid_spec = pltpu.PrefetchScalarGridSpec(
        num_scalar_prefetch=1,
        grid=(NB,),
        in_specs=[
            pl.BlockSpec((M, D), lambda i, bg: (i, 0)),
            pl.BlockSpec((1, D, D), lambda i, bg: (bg[i], 0, 0)),
            pl.BlockSpec((1, 1, D), lambda i, bg: (bg[i], 0, 0)),
        ],
        out_specs=pl.BlockSpec((M, D), lambda i, bg: (i, 0)),
    )
    return pl.pallas_call(
        body,
        grid_spec=grid_spec,
        out_shape=jax.ShapeDtypeStruct((NB * M, D), jnp.float32),
    )(block_group, x_sorted, W, b.reshape(G, 1, D))


def kernel(x, group_indices, W, b):
    B, D = x.shape
    G = W.shape[0]
    NB = B // M + G
    P = NB * M
    row_idx, block_group = _routing(group_indices, B, G, NB)
    x_sorted = _sc_gather(x, row_idx, P, D)
    y_sorted = _tc_grouped_matmul(x_sorted, W, b, block_group, NB, D)
    return _sc_scatter(y_sorted, row_idx, B, P, D)
